# baseline (speedup 1.0000x reference)
import functools
import jax
import jax.numpy as jnp
from jax import lax
from jax.experimental import pallas as pl
from jax.experimental.pallas import tpu as pltpu
from jax.experimental.pallas import tpu_sc as plsc

N_VOCAB = 1000000
NZ = 64
BATCH = 16384
CHUNK = 128
PIPE = 16

@functools.cache
def _build():
    info = plsc.get_sparse_core_info()
    nc, ns = info.num_cores, info.num_subcores
    nw = nc * ns
    b_per_w = BATCH // nw
    n_chunks = b_per_w // CHUNK
    mesh = plsc.VectorSubcoreMesh(core_axis_name="c", subcore_axis_name="s")

    @functools.partial(
        pl.kernel,
        mesh=mesh,
        out_type=jax.ShapeDtypeStruct((BATCH, NZ), jnp.float32),
        compiler_params=pltpu.CompilerParams(use_tc_tiling_on_sc=True),
        scratch_types=[
            pltpu.VMEM((b_per_w,), jnp.int32),
            pltpu.VMEM((b_per_w, NZ), jnp.float32),
        ] + [pltpu.SemaphoreType.DMA] * PIPE,
    )
    def gather_kernel(idx_hbm, table_hbm, out_hbm, idx_v, rows_v, *sems):
        wid = lax.axis_index("s") * nc + lax.axis_index("c")
        base = wid * b_per_w
        pltpu.sync_copy(idx_hbm.at[pl.ds(base, b_per_w)], idx_v)
        def fetch(r, slot):
            i = idx_v[pl.ds(r, 1)][0]
            return pltpu.async_copy(
                table_hbm.at[pl.ds(i, 1)],
                rows_v.at[pl.ds(r, 1)],
                sems[slot],
            )
        inflight = [fetch(r, r % PIPE) for r in range(PIPE)]

        def body(r, _):
            inflight[0].wait()
            del inflight[0]
            inflight.append(fetch(r, r % PIPE))
            return ()
        for r in range(PIPE, b_per_w):
            inflight[0].wait()
            del inflight[0]
            inflight.append(fetch(r, r % PIPE))
        for c in inflight:
            c.wait()
        pltpu.sync_copy(rows_v, out_hbm.at[pl.ds(base, b_per_w)])
    return gather_kernel


def kernel(idx, emb_weight):
    return _build()(idx.astype(jnp.int32), emb_weight)
